# Initial kernel scaffold; baseline (speedup 1.0000x reference)
import jax
import jax.numpy as jnp
from jax.experimental import pallas as pl


def kernel(preds, downsample_ratio, gt_shrink, gt_shrink_mask):
    raise NotImplementedError("write your pallas kernel here")


# speedup vs baseline: 18.3899x; 18.3899x over previous
import jax
import jax.numpy as jnp
from jax.experimental import pallas as pl
from jax.experimental.pallas import tpu as pltpu

_B, _H, _W = 8, 512, 512
_N = _B * _H * _W
_ROWS, _COLS = 2048, 1024
_TILE = 256
_NT = _ROWS // _TILE
_NROUNDS = 16
_NEG_RATIO = 3.0
_EPS = 1e-06

_S_POS_CNT, _S_NEG_CNT, _S_POS_LOSS, _S_C1, _S_C2, _S_C3, _S_RELU, _S_KF = range(8)
_I_LO, _I_K = range(2)


def _bce_body(x_ref, gt_ref, m_ref, out_ref, slab, smf, smi):
    r = pl.program_id(0)
    t = pl.program_id(1)

    @pl.when(r == 0)
    def _pass1():
        @pl.when(t == 0)
        def _init():
            smf[_S_POS_CNT] = 0.0
            smf[_S_NEG_CNT] = 0.0
            smf[_S_POS_LOSS] = 0.0

        x = x_ref[...]
        gt = (gt_ref[...] > 0).astype(jnp.float32)
        m = m_ref[...].astype(jnp.float32)
        loss = jnp.maximum(x, 0.0) - x * gt + jnp.log1p(jnp.exp(-jnp.abs(x)))
        pos = gt * m
        neg = (1.0 - gt) * m
        smf[_S_POS_CNT] += jnp.sum(pos)
        smf[_S_NEG_CNT] += jnp.sum(neg)
        smf[_S_POS_LOSS] += jnp.sum(loss * pos)
        slab[pl.ds(t * _TILE, _TILE), :] = loss * neg

    @pl.when((r >= 1) & (r <= _NROUNDS))
    def _bisect():
        i = r - 1
        step = jnp.maximum(jnp.int32(1), jnp.int32(1 << 29) >> (2 * i))

        @pl.when((r == 1) & (t == 0))
        def _init_k():
            pos_i = smf[_S_POS_CNT].astype(jnp.int32)
            neg_i = smf[_S_NEG_CNT].astype(jnp.int32)
            cap = (smf[_S_POS_CNT] * _NEG_RATIO).astype(jnp.int32)
            k = jnp.minimum(neg_i, cap)
            smi[_I_K] = k
            smf[_S_KF] = k.astype(jnp.float32)
            smi[_I_LO] = 0
            del pos_i

        @pl.when(t == 0)
        def _zero_counts():
            smf[_S_C1] = 0.0
            smf[_S_C2] = 0.0
            smf[_S_C3] = 0.0

        lo = smi[_I_LO]
        bits = jax.lax.bitcast_convert_type(
            slab[pl.ds(t * _TILE, _TILE), :], jnp.int32)
        smf[_S_C1] += jnp.sum((bits >= lo + step).astype(jnp.float32))
        smf[_S_C2] += jnp.sum((bits >= lo + 2 * step).astype(jnp.float32))
        smf[_S_C3] += jnp.sum((bits >= lo + 3 * step).astype(jnp.float32))

        @pl.when(t == _NT - 1)
        def _decide():
            kf = smf[_S_KF]
            jmax = ((smf[_S_C1] >= kf).astype(jnp.int32)
                    + (smf[_S_C2] >= kf).astype(jnp.int32)
                    + (smf[_S_C3] >= kf).astype(jnp.int32))
            smi[_I_LO] = lo + jmax * step

    @pl.when(r == _NROUNDS + 1)
    def _final():
        @pl.when(t == 0)
        def _zero_relu():
            smf[_S_RELU] = 0.0

        tstar = jax.lax.bitcast_convert_type(smi[_I_LO], jnp.float32)
        v = slab[pl.ds(t * _TILE, _TILE), :]
        smf[_S_RELU] += jnp.sum(jnp.maximum(v - tstar, 0.0))

        @pl.when(t == _NT - 1)
        def _assemble():
            k = smi[_I_K]
            kf = smf[_S_KF]
            neg_top = jnp.where(k > 0, smf[_S_RELU] + kf * tstar, 0.0)
            pos_i = smf[_S_POS_CNT].astype(jnp.int32)
            denom = (pos_i + k).astype(jnp.float32) + _EPS
            out_ref[0] = (smf[_S_POS_LOSS] + neg_top) / denom


def _balance_bce(pred2d, gt2d, m2d):
    return pl.pallas_call(
        _bce_body,
        grid=(_NROUNDS + 2, _NT),
        in_specs=[
            pl.BlockSpec((_TILE, _COLS),
                         lambda r, t: (jnp.where(r == 0, t, 0), 0)),
            pl.BlockSpec((_TILE, _COLS),
                         lambda r, t: (jnp.where(r == 0, t, 0), 0)),
            pl.BlockSpec((_TILE, _COLS),
                         lambda r, t: (jnp.where(r == 0, t, 0), 0)),
        ],
        out_specs=pl.BlockSpec(memory_space=pltpu.SMEM),
        out_shape=jax.ShapeDtypeStruct((1,), jnp.float32),
        scratch_shapes=[
            pltpu.VMEM((_ROWS, _COLS), jnp.float32),
            pltpu.SMEM((8,), jnp.float32),
            pltpu.SMEM((2,), jnp.int32),
        ],
        compiler_params=pltpu.CompilerParams(
            dimension_semantics=("arbitrary", "arbitrary")),
    )(pred2d, gt2d, m2d)


def kernel(preds, downsample_ratio, gt_shrink, gt_shrink_mask):
    pred2d = preds.reshape(_ROWS, _COLS)
    gt2d = gt_shrink.reshape(_ROWS, _COLS)
    m2d = gt_shrink_mask.reshape(_ROWS, _COLS)
    out = _balance_bce(pred2d, gt2d, m2d)
    return out[0] * jnp.float32(1.0) * downsample_ratio


# speedup vs baseline: 41.1473x; 2.2375x over previous
import jax
import jax.numpy as jnp
from jax.experimental import pallas as pl
from jax.experimental.pallas import tpu as pltpu

_B, _H, _W = 8, 512, 512
_N = _B * _H * _W
_ROWS, _COLS = 2048, 1024
_TILE = 256
_NT = _ROWS // _TILE
_NROUNDS = 16
_NEG_RATIO = 3.0
_EPS = 1e-06

(_S_POS_CNT, _S_MASK_CNT, _S_POS_LOSS, _S_NEG_LOSS, _S_C1, _S_C2, _S_C3,
 _S_RELU, _S_KF) = range(9)
_I_LO, _I_K, _I_SLOW = range(3)


def _bce_body(x_ref, gt_ref, m_ref, out_ref, slab, smf, smi):
    r = pl.program_id(0)
    t = pl.program_id(1)

    @pl.when(r == 0)
    def _pass1():
        @pl.when(t == 0)
        def _init():
            smf[_S_POS_CNT] = 0.0
            smf[_S_MASK_CNT] = 0.0
            smf[_S_POS_LOSS] = 0.0
            smf[_S_NEG_LOSS] = 0.0

        x = x_ref[...]
        gt = (gt_ref[...].astype(jnp.float32) > 0.0).astype(jnp.float32)
        m = m_ref[...].astype(jnp.float32)
        loss = jnp.maximum(x, 0.0) - x * gt + jnp.log1p(jnp.exp(-jnp.abs(x)))
        pos = gt * m
        neg_loss = loss * (m - pos)
        smf[_S_POS_CNT] += jnp.sum(pos)
        smf[_S_MASK_CNT] += jnp.sum(m)
        smf[_S_POS_LOSS] += jnp.sum(loss * pos)
        smf[_S_NEG_LOSS] += jnp.sum(neg_loss)
        slab[pl.ds(t * _TILE, _TILE), :] = neg_loss

    @pl.when((r == 1) & (t == 0))
    def _init_k():
        pos_f = smf[_S_POS_CNT]
        neg_i = (smf[_S_MASK_CNT] - pos_f).astype(jnp.int32)
        cap = (pos_f * _NEG_RATIO).astype(jnp.int32)
        k = jnp.minimum(neg_i, cap)
        smi[_I_K] = k
        smf[_S_KF] = k.astype(jnp.float32)
        smi[_I_LO] = 0
        smi[_I_SLOW] = (cap < neg_i).astype(jnp.int32)
        smf[_S_RELU] = 0.0

    @pl.when((r >= 1) & (r <= _NROUNDS))
    def _bisect():
        @pl.when(smi[_I_SLOW] == 1)
        def _do_round():
            i = r - 1
            step = jnp.maximum(jnp.int32(1), jnp.int32(1 << 29) >> (2 * i))

            @pl.when(t == 0)
            def _zero_counts():
                smf[_S_C1] = 0.0
                smf[_S_C2] = 0.0
                smf[_S_C3] = 0.0

            lo = smi[_I_LO]
            bits = jax.lax.bitcast_convert_type(
                slab[pl.ds(t * _TILE, _TILE), :], jnp.int32)
            smf[_S_C1] += jnp.sum((bits >= lo + step).astype(jnp.float32))
            smf[_S_C2] += jnp.sum((bits >= lo + 2 * step).astype(jnp.float32))
            smf[_S_C3] += jnp.sum((bits >= lo + 3 * step).astype(jnp.float32))

            @pl.when(t == _NT - 1)
            def _decide():
                kf = smf[_S_KF]
                jmax = ((smf[_S_C1] >= kf).astype(jnp.int32)
                        + (smf[_S_C2] >= kf).astype(jnp.int32)
                        + (smf[_S_C3] >= kf).astype(jnp.int32))
                smi[_I_LO] = lo + jmax * step

    @pl.when(r == _NROUNDS + 1)
    def _final():
        tstar = jax.lax.bitcast_convert_type(smi[_I_LO], jnp.float32)

        @pl.when(smi[_I_SLOW] == 1)
        def _relu_pass():
            v = slab[pl.ds(t * _TILE, _TILE), :]
            smf[_S_RELU] += jnp.sum(jnp.maximum(v - tstar, 0.0))

        @pl.when(t == _NT - 1)
        def _assemble():
            k = smi[_I_K]
            kf = smf[_S_KF]
            slow_top = jnp.where(k > 0, smf[_S_RELU] + kf * tstar, 0.0)
            neg_top = jnp.where(smi[_I_SLOW] == 1, slow_top,
                                smf[_S_NEG_LOSS])
            pos_i = smf[_S_POS_CNT].astype(jnp.int32)
            denom = (pos_i + k).astype(jnp.float32) + _EPS
            out_ref[0] = (smf[_S_POS_LOSS] + neg_top) / denom


def _balance_bce(pred2d, gt2d, m2d):
    return pl.pallas_call(
        _bce_body,
        grid=(_NROUNDS + 2, _NT),
        in_specs=[
            pl.BlockSpec((_TILE, _COLS),
                         lambda r, t: (jnp.where(r == 0, t, 0), 0)),
            pl.BlockSpec((_TILE, _COLS),
                         lambda r, t: (jnp.where(r == 0, t, 0), 0)),
            pl.BlockSpec((_TILE, _COLS),
                         lambda r, t: (jnp.where(r == 0, t, 0), 0)),
        ],
        out_specs=pl.BlockSpec(memory_space=pltpu.SMEM),
        out_shape=jax.ShapeDtypeStruct((1,), jnp.float32),
        scratch_shapes=[
            pltpu.VMEM((_ROWS, _COLS), jnp.float32),
            pltpu.SMEM((9,), jnp.float32),
            pltpu.SMEM((3,), jnp.int32),
        ],
        compiler_params=pltpu.CompilerParams(
            dimension_semantics=("arbitrary", "arbitrary")),
    )(pred2d, gt2d, m2d)


def kernel(preds, downsample_ratio, gt_shrink, gt_shrink_mask):
    pred2d = preds.reshape(_ROWS, _COLS)
    gt2d = gt_shrink.astype(jnp.int8).reshape(_ROWS, _COLS)
    m2d = gt_shrink_mask.astype(jnp.int8).reshape(_ROWS, _COLS)
    out = _balance_bce(pred2d, gt2d, m2d)
    return out[0] * jnp.float32(1.0) * downsample_ratio
